# baseline (speedup 1.0000x reference)
import functools

import jax
import jax.numpy as jnp
from jax import lax
from jax.experimental import pallas as pl
from jax.experimental.pallas import tpu as pltpu
from jax.experimental.pallas import tpu_sc as plsc

T = 2048
D = 768
E = 8
FF = 1024
BLK = 128
P = T * 2 + E * BLK
NB = P // BLK
NC, NS = 2, 16
NW = NC * NS
TPW = T // NW


def _fiota(shape, dim):
    return lax.broadcasted_iota(jnp.int32, shape, dim).astype(jnp.float32)


def _meta_body(x_ref, gw_ref, be_ref, p0_ref, p1_ref, w1_ref, w2_ref):
    x = x_ref[...]
    logits = lax.dot_general(gw_ref[...], x, (((1,), (1,)), ((), ())),
                             preferred_element_type=jnp.float32)
    logits = logits - jnp.max(logits, axis=0, keepdims=True)
    ex = jnp.exp(logits)
    probs = ex / jnp.sum(ex, axis=0, keepdims=True)

    eidx = _fiota((E, T), 0)
    m1 = jnp.max(probs, axis=0, keepdims=True)
    i1 = jnp.min(jnp.where(probs == m1, eidx, float(E)), axis=0, keepdims=True)
    mask1 = (eidx == i1).astype(jnp.float32)
    pr2 = jnp.where(mask1 > 0, -1.0, probs)
    m2 = jnp.max(pr2, axis=0, keepdims=True)
    i2 = jnp.min(jnp.where(pr2 == m2, eidx, float(E)), axis=0, keepdims=True)
    mask2 = (eidx == i2).astype(jnp.float32)
    denom = m1 + m2
    w1_ref[...] = m1 / denom
    w2_ref[...] = m2 / denom

    mstack = jnp.concatenate([mask1, mask2], axis=0)
    jj = _fiota((BLK, BLK), 0)
    ii = _fiota((BLK, BLK), 1)
    tri = (jj <= ii).astype(jnp.float32)
    cols = []
    carry = jnp.zeros((2 * E, 1), jnp.float32)
    for q in range(T // BLK):
        mq = mstack[:, q * BLK:(q + 1) * BLK]
        cq = lax.dot_general(mq, tri, (((1,), (0,)), ((), ())),
                             preferred_element_type=jnp.float32) + carry
        carry = cq[:, BLK - 1:BLK]
        cols.append(cq)
    call = jnp.concatenate(cols, axis=1)
    c0 = call[:E]
    c1 = call[E:] + c0[:, T - 1:T]
    cnt = c1[:, T - 1:T]
    pcnt = jnp.floor((cnt + float(BLK - 1)) * (1.0 / BLK)) * float(BLK)

    ae = _fiota((E, E), 0)
    ap = _fiota((E, E), 1)
    strict = (ap < ae).astype(jnp.float32)
    off = lax.dot_general(strict, pcnt, (((1,), (0,)), ((), ())),
                          preferred_element_type=jnp.float32)

    r0 = jnp.sum(mask1 * (c0 - 1.0), axis=0, keepdims=True)
    r1 = jnp.sum(mask2 * (c1 - 1.0), axis=0, keepdims=True)
    o0 = jnp.sum(mask1 * off, axis=0, keepdims=True)
    o1 = jnp.sum(mask2 * off, axis=0, keepdims=True)
    p0_ref[...] = (o0 + r0).astype(jnp.int32)
    p1_ref[...] = (o1 + r1).astype(jnp.int32)

    starts = float(BLK) * _fiota((1, NB), 1)
    e8 = _fiota((E, NB), 0)
    inb = jnp.logical_and(starts >= off, starts < off + pcnt)
    be = jnp.sum(jnp.where(inb, e8, 0.0), axis=0, keepdims=True)
    be_ref[...] = be.astype(jnp.int32)


def _route_meta(x, gate_w):
    return pl.pallas_call(
        _meta_body,
        out_shape=(
            jax.ShapeDtypeStruct((1, NB), jnp.int32),
            jax.ShapeDtypeStruct((1, T), jnp.int32),
            jax.ShapeDtypeStruct((1, T), jnp.int32),
            jax.ShapeDtypeStruct((1, T), jnp.float32),
            jax.ShapeDtypeStruct((1, T), jnp.float32),
        ),
    )(x, gate_w)


def _ffn_body(be_ref, p0_ref, p1_ref, w1_ref, w2_ref, x_ref, wg_ref, wu_ref,
              wd_ref, ys_ref):
    b = pl.program_id(0)
    pos = float(BLK) * b.astype(jnp.float32) + _fiota((BLK, 1), 0)
    p0 = p0_ref[...].astype(jnp.float32)
    p1 = p1_ref[...].astype(jnp.float32)
    e0 = (p0 == pos).astype(jnp.float32)
    e1 = (p1 == pos).astype(jnp.float32)
    oneh = e0 + e1
    xrows = lax.dot_general(oneh, x_ref[...], (((1,), (0,)), ((), ())),
                            preferred_element_type=jnp.float32
                            ).astype(jnp.bfloat16)
    w = jnp.sum(e0 * w1_ref[...] + e1 * w2_ref[...], axis=1, keepdims=True)

    g = lax.dot_general(xrows, wg_ref[0], (((1,), (1,)), ((), ())),
                        preferred_element_type=jnp.float32)
    u = lax.dot_general(xrows, wu_ref[0], (((1,), (1,)), ((), ())),
                        preferred_element_type=jnp.float32)
    act = (g * jax.nn.sigmoid(g) * u).astype(jnp.bfloat16)
    o = lax.dot_general(act, wd_ref[0], (((1,), (1,)), ((), ())),
                        preferred_element_type=jnp.float32)
    ys_ref[...] = o * w


def _grouped_ffn(be, p0, p1, w1, w2, x, w_gate, w_up, w_down):
    grid_spec = pltpu.PrefetchScalarGridSpec(
        num_scalar_prefetch=1,
        grid=(NB,),
        in_specs=[
            pl.BlockSpec((1, T), lambda b, be: (0, 0)),
            pl.BlockSpec((1, T), lambda b, be: (0, 0)),
            pl.BlockSpec((1, T), lambda b, be: (0, 0)),
            pl.BlockSpec((1, T), lambda b, be: (0, 0)),
            pl.BlockSpec((T, D), lambda b, be: (0, 0)),
            pl.BlockSpec((1, FF, D), lambda b, be: (be[b], 0, 0)),
            pl.BlockSpec((1, FF, D), lambda b, be: (be[b], 0, 0)),
            pl.BlockSpec((1, D, FF), lambda b, be: (be[b], 0, 0)),
        ],
        out_specs=pl.BlockSpec((BLK, D), lambda b, be: (b, 0)),
    )
    return pl.pallas_call(
        _ffn_body,
        grid_spec=grid_spec,
        out_shape=jax.ShapeDtypeStruct((P, D), jnp.float32),
    )(be, p0, p1, w1, w2, x, w_gate, w_up, w_down)


@functools.cache
def _make_sc_combine():
    @functools.partial(
        pl.kernel,
        out_type=jax.ShapeDtypeStruct((T, D), jnp.float32),
        mesh=plsc.VectorSubcoreMesh(core_axis_name="c", subcore_axis_name="s"),
        scratch_types=[
            pltpu.VMEM((TPW,), jnp.int32),
            pltpu.VMEM((TPW,), jnp.int32),
            pltpu.VMEM((TPW, D), jnp.float32),
            pltpu.VMEM((TPW, D), jnp.float32),
            pltpu.SemaphoreType.DMA,
            pltpu.SemaphoreType.DMA,
        ],
    )
    def _sc_combine(ys_hbm, p0_hbm, p1_hbm, y_hbm, i0_v, i1_v, b0, b1, s0,
                    s1):
        wid = lax.axis_index("s") * NC + lax.axis_index("c")
        base = wid * TPW
        pltpu.sync_copy(p0_hbm.at[pl.ds(base, TPW)], i0_v)
        pltpu.sync_copy(p1_hbm.at[pl.ds(base, TPW)], i1_v)
        d0 = pltpu.async_copy(ys_hbm.at[i0_v], b0, s0)
        d1 = pltpu.async_copy(ys_hbm.at[i1_v], b1, s1)
        d0.wait()
        d1.wait()

        def _row(j, carry):
            for k in range(D // 16):
                sl = pl.ds(k * 16, 16)
                b0[j, sl] = b0[j, sl] + b1[j, sl]
            return carry

        lax.fori_loop(0, TPW, _row, 0)
        pltpu.sync_copy(b0, y_hbm.at[pl.ds(base, TPW)])

    return _sc_combine


@jax.jit
def kernel(hidden_states, gate_w, w_gate, w_up, w_down):
    be, p0, p1, w1, w2 = _route_meta(hidden_states, gate_w)
    ys = _grouped_ffn(be.reshape(NB), p0, p1, w1, w2, hidden_states,
                      w_gate, w_up, w_down)
    return _make_sc_combine()(ys, p0.reshape(T), p1.reshape(T))


# speedup vs baseline: 1.2416x; 1.2416x over previous
import functools

import jax
import jax.numpy as jnp
from jax import lax
from jax.experimental import pallas as pl
from jax.experimental.pallas import tpu as pltpu
from jax.experimental.pallas import tpu_sc as plsc

T = 2048
D = 768
E = 8
FF = 1024
BLK = 256
P = T * 2 + E * BLK
NB = P // BLK
NC, NS = 2, 16
NW = NC * NS
TPW = T // NW


def _fiota(shape, dim):
    return lax.broadcasted_iota(jnp.int32, shape, dim).astype(jnp.float32)


def _meta_body(x_ref, gw_ref, be_ref, p0_ref, p1_ref, w1_ref, w2_ref):
    x = x_ref[...]
    logits = lax.dot_general(gw_ref[...], x, (((1,), (1,)), ((), ())),
                             preferred_element_type=jnp.float32)
    logits = logits - jnp.max(logits, axis=0, keepdims=True)
    ex = jnp.exp(logits)
    probs = ex / jnp.sum(ex, axis=0, keepdims=True)

    eidx = _fiota((E, T), 0)
    m1 = jnp.max(probs, axis=0, keepdims=True)
    i1 = jnp.min(jnp.where(probs == m1, eidx, float(E)), axis=0, keepdims=True)
    mask1 = (eidx == i1).astype(jnp.float32)
    pr2 = jnp.where(mask1 > 0, -1.0, probs)
    m2 = jnp.max(pr2, axis=0, keepdims=True)
    i2 = jnp.min(jnp.where(pr2 == m2, eidx, float(E)), axis=0, keepdims=True)
    mask2 = (eidx == i2).astype(jnp.float32)
    denom = m1 + m2
    w1_ref[...] = m1 / denom
    w2_ref[...] = m2 / denom

    mstack = jnp.concatenate([mask1, mask2], axis=0)
    jj = _fiota((BLK, BLK), 0)
    ii = _fiota((BLK, BLK), 1)
    tri = (jj <= ii).astype(jnp.float32)
    cols = []
    carry = jnp.zeros((2 * E, 1), jnp.float32)
    for q in range(T // BLK):
        mq = mstack[:, q * BLK:(q + 1) * BLK]
        cq = lax.dot_general(mq, tri, (((1,), (0,)), ((), ())),
                             preferred_element_type=jnp.float32) + carry
        carry = cq[:, BLK - 1:BLK]
        cols.append(cq)
    call = jnp.concatenate(cols, axis=1)
    c0 = call[:E]
    c1 = call[E:] + c0[:, T - 1:T]
    cnt = c1[:, T - 1:T]
    pcnt = jnp.floor((cnt + float(BLK - 1)) * (1.0 / BLK)) * float(BLK)

    ae = _fiota((E, E), 0)
    ap = _fiota((E, E), 1)
    strict = (ap < ae).astype(jnp.float32)
    off = lax.dot_general(strict, pcnt, (((1,), (0,)), ((), ())),
                          preferred_element_type=jnp.float32)

    r0 = jnp.sum(mask1 * (c0 - 1.0), axis=0, keepdims=True)
    r1 = jnp.sum(mask2 * (c1 - 1.0), axis=0, keepdims=True)
    o0 = jnp.sum(mask1 * off, axis=0, keepdims=True)
    o1 = jnp.sum(mask2 * off, axis=0, keepdims=True)
    p0_ref[...] = (o0 + r0).astype(jnp.int32)
    p1_ref[...] = (o1 + r1).astype(jnp.int32)

    starts = float(BLK) * _fiota((1, NB), 1)
    e8 = _fiota((E, NB), 0)
    inb = jnp.logical_and(starts >= off, starts < off + pcnt)
    be = jnp.sum(jnp.where(inb, e8, 0.0), axis=0, keepdims=True)
    be_ref[...] = be.astype(jnp.int32)


def _route_meta(x, gate_w):
    return pl.pallas_call(
        _meta_body,
        out_shape=(
            jax.ShapeDtypeStruct((1, NB), jnp.int32),
            jax.ShapeDtypeStruct((1, T), jnp.int32),
            jax.ShapeDtypeStruct((1, T), jnp.int32),
            jax.ShapeDtypeStruct((1, T), jnp.float32),
            jax.ShapeDtypeStruct((1, T), jnp.float32),
        ),
    )(x, gate_w)


def _ffn_body(be_ref, p0_ref, p1_ref, w1_ref, w2_ref, x_ref, wg_ref, wu_ref,
              wd_ref, ys_ref):
    b = pl.program_id(0)
    pos = float(BLK) * b.astype(jnp.float32) + _fiota((BLK, 1), 0)
    p0 = p0_ref[...].astype(jnp.float32)
    p1 = p1_ref[...].astype(jnp.float32)
    e0 = (p0 == pos).astype(jnp.float32)
    e1 = (p1 == pos).astype(jnp.float32)
    oneh = e0 + e1
    xrows = lax.dot_general(oneh, x_ref[...], (((1,), (0,)), ((), ())),
                            preferred_element_type=jnp.float32)
    w = jnp.sum(e0 * w1_ref[...] + e1 * w2_ref[...], axis=1, keepdims=True)

    g = lax.dot_general(xrows, wg_ref[0], (((1,), (1,)), ((), ())),
                        preferred_element_type=jnp.float32)
    u = lax.dot_general(xrows, wu_ref[0], (((1,), (1,)), ((), ())),
                        preferred_element_type=jnp.float32)
    act = g * jax.nn.sigmoid(g) * u
    o = lax.dot_general(act, wd_ref[0], (((1,), (1,)), ((), ())),
                        preferred_element_type=jnp.float32)
    ys_ref[...] = o * w


def _grouped_ffn(be, p0, p1, w1, w2, x, w_gate, w_up, w_down):
    grid_spec = pltpu.PrefetchScalarGridSpec(
        num_scalar_prefetch=1,
        grid=(NB,),
        in_specs=[
            pl.BlockSpec((1, T), lambda b, be: (0, 0)),
            pl.BlockSpec((1, T), lambda b, be: (0, 0)),
            pl.BlockSpec((1, T), lambda b, be: (0, 0)),
            pl.BlockSpec((1, T), lambda b, be: (0, 0)),
            pl.BlockSpec((T, D), lambda b, be: (0, 0)),
            pl.BlockSpec((1, FF, D), lambda b, be: (be[b], 0, 0)),
            pl.BlockSpec((1, FF, D), lambda b, be: (be[b], 0, 0)),
            pl.BlockSpec((1, D, FF), lambda b, be: (be[b], 0, 0)),
        ],
        out_specs=pl.BlockSpec((BLK, D), lambda b, be: (b, 0)),
    )
    return pl.pallas_call(
        _ffn_body,
        grid_spec=grid_spec,
        out_shape=jax.ShapeDtypeStruct((P, D), jnp.float32),
    )(be, p0, p1, w1, w2, x, w_gate, w_up, w_down)


@functools.cache
def _make_sc_combine():
    @functools.partial(
        pl.kernel,
        out_type=jax.ShapeDtypeStruct((T, D), jnp.float32),
        mesh=plsc.VectorSubcoreMesh(core_axis_name="c", subcore_axis_name="s"),
        scratch_types=[
            pltpu.VMEM((TPW,), jnp.int32),
            pltpu.VMEM((TPW,), jnp.int32),
            pltpu.VMEM((TPW, D), jnp.float32),
            pltpu.VMEM((TPW, D), jnp.float32),
            pltpu.SemaphoreType.DMA,
            pltpu.SemaphoreType.DMA,
        ],
    )
    def _sc_combine(ys_hbm, p0_hbm, p1_hbm, y_hbm, i0_v, i1_v, b0, b1, s0,
                    s1):
        wid = lax.axis_index("s") * NC + lax.axis_index("c")
        base = wid * TPW
        pltpu.sync_copy(p0_hbm.at[pl.ds(base, TPW)], i0_v)
        pltpu.sync_copy(p1_hbm.at[pl.ds(base, TPW)], i1_v)
        d0 = pltpu.async_copy(ys_hbm.at[i0_v], b0, s0)
        d1 = pltpu.async_copy(ys_hbm.at[i1_v], b1, s1)
        d0.wait()
        d1.wait()

        def _row(j, carry):
            for k in range(D // 16):
                sl = pl.ds(k * 16, 16)
                b0[j, sl] = b0[j, sl] + b1[j, sl]
            return carry

        lax.fori_loop(0, TPW, _row, 0)
        pltpu.sync_copy(b0, y_hbm.at[pl.ds(base, TPW)])

    return _sc_combine


@jax.jit
def kernel(hidden_states, gate_w, w_gate, w_up, w_down):
    be, p0, p1, w1, w2 = _route_meta(hidden_states, gate_w)
    ys = _grouped_ffn(be.reshape(NB), p0, p1, w1, w2, hidden_states,
                      w_gate, w_up, w_down)
    return _make_sc_combine()(ys, p0.reshape(T), p1.reshape(T))


# speedup vs baseline: 1.3581x; 1.0938x over previous
import functools

import jax
import jax.numpy as jnp
from jax import lax
from jax.experimental import pallas as pl
from jax.experimental.pallas import tpu as pltpu
from jax.experimental.pallas import tpu_sc as plsc

T = 2048
D = 768
E = 8
FF = 1024
BLK = 256
P = T * 2 + E * BLK
NB = P // BLK
NC, NS = 2, 16
NW = NC * NS
TPW = T // NW


def _fiota(shape, dim):
    return lax.broadcasted_iota(jnp.int32, shape, dim).astype(jnp.float32)


def _meta_body(x_ref, gw_ref, be_ref, act_ref, p0_ref, p1_ref, w1_ref, w2_ref):
    x = x_ref[...]
    logits = lax.dot_general(gw_ref[...], x, (((1,), (1,)), ((), ())),
                             preferred_element_type=jnp.float32)
    logits = logits - jnp.max(logits, axis=0, keepdims=True)
    ex = jnp.exp(logits)
    probs = ex / jnp.sum(ex, axis=0, keepdims=True)

    eidx = _fiota((E, T), 0)
    m1 = jnp.max(probs, axis=0, keepdims=True)
    i1 = jnp.min(jnp.where(probs == m1, eidx, float(E)), axis=0, keepdims=True)
    mask1 = (eidx == i1).astype(jnp.float32)
    pr2 = jnp.where(mask1 > 0, -1.0, probs)
    m2 = jnp.max(pr2, axis=0, keepdims=True)
    i2 = jnp.min(jnp.where(pr2 == m2, eidx, float(E)), axis=0, keepdims=True)
    mask2 = (eidx == i2).astype(jnp.float32)
    denom = m1 + m2
    w1_ref[...] = m1 / denom
    w2_ref[...] = m2 / denom

    mstack = jnp.concatenate([mask1, mask2], axis=0)
    jj = _fiota((BLK, BLK), 0)
    ii = _fiota((BLK, BLK), 1)
    tri = (jj <= ii).astype(jnp.float32)
    cols = []
    carry = jnp.zeros((2 * E, 1), jnp.float32)
    for q in range(T // BLK):
        mq = mstack[:, q * BLK:(q + 1) * BLK]
        cq = lax.dot_general(mq, tri, (((1,), (0,)), ((), ())),
                             preferred_element_type=jnp.float32) + carry
        carry = cq[:, BLK - 1:BLK]
        cols.append(cq)
    call = jnp.concatenate(cols, axis=1)
    c0 = call[:E]
    c1 = call[E:] + c0[:, T - 1:T]
    cnt = c1[:, T - 1:T]
    pcnt = jnp.floor((cnt + float(BLK - 1)) * (1.0 / BLK)) * float(BLK)

    ae = _fiota((E, E), 0)
    ap = _fiota((E, E), 1)
    strict = (ap < ae).astype(jnp.float32)
    off = lax.dot_general(strict, pcnt, (((1,), (0,)), ((), ())),
                          preferred_element_type=jnp.float32)

    r0 = jnp.sum(mask1 * (c0 - 1.0), axis=0, keepdims=True)
    r1 = jnp.sum(mask2 * (c1 - 1.0), axis=0, keepdims=True)
    o0 = jnp.sum(mask1 * off, axis=0, keepdims=True)
    o1 = jnp.sum(mask2 * off, axis=0, keepdims=True)
    p0_ref[...] = (o0 + r0).astype(jnp.int32)
    p1_ref[...] = (o1 + r1).astype(jnp.int32)

    starts = float(BLK) * _fiota((1, NB), 1)
    ends = off + pcnt
    be = jnp.sum((starts >= ends).astype(jnp.float32), axis=0, keepdims=True)
    be_ref[...] = jnp.minimum(be, float(E - 1)).astype(jnp.int32)
    total = jnp.max(ends, axis=0, keepdims=True)
    act_ref[...] = (starts < total).astype(jnp.int32)


def _route_meta(x, gate_w):
    return pl.pallas_call(
        _meta_body,
        out_shape=(
            jax.ShapeDtypeStruct((1, NB), jnp.int32),
            jax.ShapeDtypeStruct((1, NB), jnp.int32),
            jax.ShapeDtypeStruct((1, T), jnp.int32),
            jax.ShapeDtypeStruct((1, T), jnp.int32),
            jax.ShapeDtypeStruct((1, T), jnp.float32),
            jax.ShapeDtypeStruct((1, T), jnp.float32),
        ),
    )(x, gate_w)


def _ffn_body(be_ref, act_ref, p0_ref, p1_ref, w1_ref, w2_ref, x_ref, wg_ref,
              wu_ref, wd_ref, ys_ref):
    b = pl.program_id(0)

    @pl.when(act_ref[b] == 1)
    def _active():
        pos = float(BLK) * b.astype(jnp.float32) + _fiota((BLK, 1), 0)
        p0 = p0_ref[...].astype(jnp.float32)
        p1 = p1_ref[...].astype(jnp.float32)
        e0 = (p0 == pos).astype(jnp.float32)
        e1 = (p1 == pos).astype(jnp.float32)
        oneh = e0 + e1
        xrows = lax.dot_general(oneh, x_ref[...], (((1,), (0,)), ((), ())),
                                preferred_element_type=jnp.float32)
        w = jnp.sum(e0 * w1_ref[...] + e1 * w2_ref[...], axis=1,
                    keepdims=True)

        g = lax.dot_general(xrows, wg_ref[0], (((1,), (1,)), ((), ())),
                            preferred_element_type=jnp.float32)
        u = lax.dot_general(xrows, wu_ref[0], (((1,), (1,)), ((), ())),
                            preferred_element_type=jnp.float32)
        act = g * jax.nn.sigmoid(g) * u
        o = lax.dot_general(act, wd_ref[0], (((1,), (1,)), ((), ())),
                            preferred_element_type=jnp.float32)
        ys_ref[...] = o * w


def _grouped_ffn(be, act, p0, p1, w1, w2, x, w_gate, w_up, w_down):
    grid_spec = pltpu.PrefetchScalarGridSpec(
        num_scalar_prefetch=2,
        grid=(NB,),
        in_specs=[
            pl.BlockSpec((1, T), lambda b, be, act: (0, 0)),
            pl.BlockSpec((1, T), lambda b, be, act: (0, 0)),
            pl.BlockSpec((1, T), lambda b, be, act: (0, 0)),
            pl.BlockSpec((1, T), lambda b, be, act: (0, 0)),
            pl.BlockSpec((T, D), lambda b, be, act: (0, 0)),
            pl.BlockSpec((1, FF, D), lambda b, be, act: (be[b], 0, 0)),
            pl.BlockSpec((1, FF, D), lambda b, be, act: (be[b], 0, 0)),
            pl.BlockSpec((1, D, FF), lambda b, be, act: (be[b], 0, 0)),
        ],
        out_specs=pl.BlockSpec((BLK, D), lambda b, be, act: (b, 0)),
    )
    return pl.pallas_call(
        _ffn_body,
        grid_spec=grid_spec,
        out_shape=jax.ShapeDtypeStruct((P, D), jnp.float32),
    )(be, act, p0, p1, w1, w2, x, w_gate, w_up, w_down)


@functools.cache
def _make_sc_combine():
    @functools.partial(
        pl.kernel,
        out_type=jax.ShapeDtypeStruct((T, D), jnp.float32),
        mesh=plsc.VectorSubcoreMesh(core_axis_name="c", subcore_axis_name="s"),
        scratch_types=[
            pltpu.VMEM((TPW,), jnp.int32),
            pltpu.VMEM((TPW,), jnp.int32),
            pltpu.VMEM((TPW, D), jnp.float32),
            pltpu.VMEM((TPW, D), jnp.float32),
            pltpu.SemaphoreType.DMA,
            pltpu.SemaphoreType.DMA,
        ],
    )
    def _sc_combine(ys_hbm, p0_hbm, p1_hbm, y_hbm, i0_v, i1_v, b0, b1, s0,
                    s1):
        wid = lax.axis_index("s") * NC + lax.axis_index("c")
        base = wid * TPW
        pltpu.sync_copy(p0_hbm.at[pl.ds(base, TPW)], i0_v)
        pltpu.sync_copy(p1_hbm.at[pl.ds(base, TPW)], i1_v)
        d0 = pltpu.async_copy(ys_hbm.at[i0_v], b0, s0)
        d1 = pltpu.async_copy(ys_hbm.at[i1_v], b1, s1)
        d0.wait()
        d1.wait()

        def _row(j, carry):
            for k in range(D // 16):
                sl = pl.ds(k * 16, 16)
                b0[j, sl] = b0[j, sl] + b1[j, sl]
            return carry

        lax.fori_loop(0, TPW, _row, 0)
        pltpu.sync_copy(b0, y_hbm.at[pl.ds(base, TPW)])

    return _sc_combine


@jax.jit
def kernel(hidden_states, gate_w, w_gate, w_up, w_down):
    be, blk_act, p0, p1, w1, w2 = _route_meta(hidden_states, gate_w)
    ys = _grouped_ffn(be.reshape(NB), blk_act.reshape(NB), p0, p1, w1, w2,
                      hidden_states, w_gate, w_up, w_down)
    return _make_sc_combine()(ys, p0.reshape(T), p1.reshape(T))
